# baseline (speedup 1.0000x reference)
import functools

import jax
import jax.numpy as jnp
from jax import lax
from jax.experimental import pallas as pl
from jax.experimental.pallas import tpu as pltpu
from jax.experimental.pallas import tpu_sc as plsc

N_NODES = 100_000
N_EDGES = 6_400_000
TABLE_ROWS = 576
NUM_WORKERS = 32
CHUNK = 2_048
NUM_CHUNKS = N_EDGES // CHUNK
SLOTS = 98
L = 16
NACC = 4
UNROLL = 2


def _sc_graph_counter(node_type, edge_type, edge_index, w_flat):
    mesh = plsc.VectorSubcoreMesh(core_axis_name="c", subcore_axis_name="s")

    @functools.partial(
        pl.kernel,
        mesh=mesh,
        out_type=jax.ShapeDtypeStruct((NUM_WORKERS, L), jnp.float32),
        compiler_params=pltpu.CompilerParams(needs_layout_passes=False),
        scratch_types=[
            pltpu.VMEM((N_NODES,), jnp.int32),
            pltpu.VMEM((TABLE_ROWS,), jnp.float32),
            pltpu.VMEM((CHUNK,), jnp.int32),
            pltpu.VMEM((CHUNK,), jnp.int32),
            pltpu.VMEM((2, CHUNK), jnp.int32),
            pltpu.VMEM((2, CHUNK), jnp.int32),
            pltpu.VMEM((L,), jnp.float32),
            pltpu.VMEM((TABLE_ROWS,), jnp.float32),
            pltpu.SemaphoreType.DMA,
            pltpu.SemaphoreType.DMA,
        ],
    )
    def k(nt_hbm, et_hbm, ei_hbm, w_hbm, out_hbm,
          nt_v, w_v, et0, et1, sd0, sd1, acc_v, hist_v, sem0, sem1):
        wid = lax.axis_index("s") * 2 + lax.axis_index("c")
        pltpu.sync_copy(nt_hbm, nt_v)
        pltpu.sync_copy(w_hbm, w_v)
        sems = (sem0, sem1)
        bufs = ((et0, sd0), (et1, sd1))

        def chunk_id(slot):
            return jnp.minimum(wid + slot * NUM_WORKERS, NUM_CHUNKS - 1)

        def start(slot, b):
            base = chunk_id(slot) * CHUNK
            et_b, sd_b = bufs[b]
            pltpu.async_copy(et_hbm.at[pl.ds(base, CHUNK)], et_b, sems[b])
            pltpu.async_copy(ei_hbm.at[:, pl.ds(base, CHUNK)], sd_b, sems[b])

        def wait(b):
            et_b, sd_b = bufs[b]
            pltpu.make_async_copy(et_hbm.at[pl.ds(0, CHUNK)], et_b,
                                  sems[b]).wait()
            pltpu.make_async_copy(ei_hbm.at[:, pl.ds(0, CHUNK)], sd_b,
                                  sems[b]).wait()

        lane = lax.iota(jnp.int32, L)
        ones_f = jnp.ones((L,), jnp.float32)

        zero = jnp.zeros((L,), jnp.float32)
        for j in range(TABLE_ROWS // L):
            hist_v[pl.ds(j * L, L)] = zero

        def compute(slot, b):
            et_b, sd_b = bufs[b]
            valid = (chunk_id(slot) == wid + slot * NUM_WORKERS)
            vmask = jnp.broadcast_to(valid, (L,))

            def vbody(i):
                for u in range(NACC):
                    j = i * NACC + u
                    sl = pl.ds(j * L, L)
                    col = j * L + lane
                    et = et_b[sl]
                    s = plsc.load_gather(sd_b, [jnp.zeros((L,), jnp.int32),
                                                col])
                    d = plsc.load_gather(sd_b, [jnp.ones((L,), jnp.int32),
                                                col])
                    a = plsc.load_gather(nt_v, [s])
                    bb = plsc.load_gather(nt_v, [d])
                    enc = et * 9 + a * 3 + bb
                    plsc.addupdate_scatter(hist_v, [enc], ones_f, mask=vmask)

            plsc.parallel_loop(
                0, (CHUNK // L) // NACC, unroll=UNROLL)(vbody)

        start(0, 0)
        start(1, 1)

        def pair_body(kk, carry):
            slot = kk * 2
            wait(0)
            compute(slot, 0)
            start(slot + 2, 0)
            wait(1)
            compute(slot + 1, 1)
            start(slot + 3, 1)
            return carry

        lax.fori_loop(0, SLOTS // 2, pair_body, jnp.int32(0))
        wait(0)
        wait(1)

        def dot_body(j, acc):
            sl = pl.ds(j * L, L)
            return acc + hist_v[sl] * w_v[sl]

        acc = lax.fori_loop(0, TABLE_ROWS // L, dot_body, zero)
        acc_v[...] = acc
        pltpu.sync_copy(acc_v, out_hbm.at[wid])

    return k(node_type, edge_type, edge_index, w_flat)


def kernel(node_type, edge_type, edge_index, scorer_weight):
    w_flat = scorer_weight.reshape(TABLE_ROWS)
    partials = _sc_graph_counter(node_type, edge_type, edge_index, w_flat)
    return jnp.sum(partials).reshape(1, 1)


# speedup vs baseline: 1.1057x; 1.1057x over previous
import functools

import jax
import jax.numpy as jnp
from jax import lax
from jax.experimental import pallas as pl
from jax.experimental.pallas import tpu as pltpu
from jax.experimental.pallas import tpu_sc as plsc

N_NODES = 100_000
N_EDGES = 6_400_000
TABLE_ROWS = 576
NUM_WORKERS = 32
CHUNK = 2_048
NUM_CHUNKS = N_EDGES // CHUNK
SLOTS = 100
NBUF = 4
L = 16
NACC = 4
UNROLL = 2


def _sc_graph_counter(node_type, edge_type, edge_index, w_flat):
    mesh = plsc.VectorSubcoreMesh(core_axis_name="c", subcore_axis_name="s")

    @functools.partial(
        pl.kernel,
        mesh=mesh,
        out_type=jax.ShapeDtypeStruct((NUM_WORKERS, L), jnp.float32),
        compiler_params=pltpu.CompilerParams(needs_layout_passes=False),
        scratch_types=[
            pltpu.VMEM((N_NODES,), jnp.int32),
            pltpu.VMEM((TABLE_ROWS,), jnp.float32),
            *([pltpu.VMEM((CHUNK,), jnp.int32)] * NBUF),
            *([pltpu.VMEM((2, CHUNK), jnp.int32)] * NBUF),
            pltpu.VMEM((L,), jnp.float32),
            *([pltpu.SemaphoreType.DMA] * NBUF),
        ],
    )
    def k(nt_hbm, et_hbm, ei_hbm, w_hbm, out_hbm, nt_v, w_v, *rest):
        et_bufs = rest[:NBUF]
        sd_bufs = rest[NBUF:2 * NBUF]
        acc_v = rest[2 * NBUF]
        sems = rest[2 * NBUF + 1:]
        bufs = tuple(zip(et_bufs, sd_bufs))
        wid = lax.axis_index("s") * 2 + lax.axis_index("c")
        pltpu.sync_copy(nt_hbm, nt_v)
        pltpu.sync_copy(w_hbm, w_v)

        def chunk_id(slot):
            return jnp.minimum(wid + slot * NUM_WORKERS, NUM_CHUNKS - 1)

        def start(slot, b):
            base = chunk_id(slot) * CHUNK
            et_b, sd_b = bufs[b]
            pltpu.async_copy(et_hbm.at[pl.ds(base, CHUNK)], et_b, sems[b])
            pltpu.async_copy(ei_hbm.at[:, pl.ds(base, CHUNK)], sd_b, sems[b])

        def wait(b):
            et_b, sd_b = bufs[b]
            pltpu.make_async_copy(et_hbm.at[pl.ds(0, CHUNK)], et_b,
                                  sems[b]).wait()
            pltpu.make_async_copy(ei_hbm.at[:, pl.ds(0, CHUNK)], sd_b,
                                  sems[b]).wait()

        lane = lax.iota(jnp.int32, L)

        def compute(slot, b, accs):
            et_b, sd_b = bufs[b]
            valid = (chunk_id(slot) == wid + slot * NUM_WORKERS)
            vmask = jnp.where(valid, 1.0, 0.0).astype(jnp.float32)
            vmask = jnp.broadcast_to(vmask, (L,))

            def vbody(i, accs):
                out = []
                for u in range(NACC):
                    j = i * NACC + u
                    sl = pl.ds(j * L, L)
                    col = j * L + lane
                    et = et_b[sl]
                    s = plsc.load_gather(sd_b, [jnp.zeros((L,), jnp.int32),
                                                col])
                    d = plsc.load_gather(sd_b, [jnp.ones((L,), jnp.int32),
                                                col])
                    a = plsc.load_gather(nt_v, [s])
                    bb = plsc.load_gather(nt_v, [d])
                    enc = et * 9 + a * 3 + bb
                    wv = plsc.load_gather(w_v, [enc])
                    out.append(accs[u] + wv * vmask)
                return tuple(out)

            return plsc.parallel_loop(
                0, (CHUNK // L) // NACC, carry=accs, unroll=UNROLL)(vbody)

        for b in range(NBUF):
            start(b, b)

        def ring_body(kk, accs):
            slot = kk * NBUF
            for b in range(NBUF):
                wait(b)
                accs = compute(slot + b, b, accs)
                start(slot + b + NBUF, b)
            return accs

        zero = jnp.zeros((L,), jnp.float32)
        accs = lax.fori_loop(0, SLOTS // NBUF, ring_body, (zero,) * NACC)
        for b in range(NBUF):
            wait(b)
        acc = accs[0]
        for u in range(1, NACC):
            acc = acc + accs[u]
        acc_v[...] = acc
        pltpu.sync_copy(acc_v, out_hbm.at[wid])

    return k(node_type, edge_type, edge_index, w_flat)


def kernel(node_type, edge_type, edge_index, scorer_weight):
    w_flat = scorer_weight.reshape(TABLE_ROWS)
    partials = _sc_graph_counter(node_type, edge_type, edge_index, w_flat)
    return jnp.sum(partials).reshape(1, 1)


# speedup vs baseline: 1.2540x; 1.1341x over previous
import functools

import jax
import jax.numpy as jnp
from jax import lax
from jax.experimental import pallas as pl
from jax.experimental.pallas import tpu as pltpu
from jax.experimental.pallas import tpu_sc as plsc

N_NODES = 100_000
N_EDGES = 6_400_000
TABLE_ROWS = 576
NUM_WORKERS = 32
CHUNK = 2_048
NUM_CHUNKS = N_EDGES // CHUNK
SLOTS = 99
NBUF = 3
L = 16
NACC = 4
UNROLL = 2


def _sc_graph_counter(node_type, edge_type, edge_index, w_flat):
    mesh = plsc.VectorSubcoreMesh(core_axis_name="c", subcore_axis_name="s")

    @functools.partial(
        pl.kernel,
        mesh=mesh,
        out_type=jax.ShapeDtypeStruct((NUM_WORKERS, L), jnp.float32),
        compiler_params=pltpu.CompilerParams(needs_layout_passes=False),
        scratch_types=[
            pltpu.VMEM((N_NODES,), jnp.int32),
            pltpu.VMEM((TABLE_ROWS,), jnp.float32),
            *([pltpu.VMEM((CHUNK,), jnp.int32)] * NBUF),
            *([pltpu.VMEM((2, CHUNK), jnp.int32)] * NBUF),
            pltpu.VMEM((L,), jnp.float32),
            pltpu.VMEM_SHARED((N_NODES,), jnp.int32),
            *([pltpu.SemaphoreType.DMA] * NBUF),
        ],
    )
    def k(nt_hbm, et_hbm, ei_hbm, w_hbm, out_hbm, nt_v, w_v, *rest):
        et_bufs = rest[:NBUF]
        sd_bufs = rest[NBUF:2 * NBUF]
        acc_v = rest[2 * NBUF]
        nt_sp = rest[2 * NBUF + 1]
        sems = rest[2 * NBUF + 2:]
        bufs = tuple(zip(et_bufs, sd_bufs))
        sid = lax.axis_index("s")
        wid = sid * 2 + lax.axis_index("c")

        def chunk_id(slot):
            return jnp.minimum(wid + slot * NUM_WORKERS, NUM_CHUNKS - 1)

        def start(slot, b):
            base = chunk_id(slot) * CHUNK
            et_b, sd_b = bufs[b]
            pltpu.async_copy(et_hbm.at[pl.ds(base, CHUNK)], et_b, sems[b])
            pltpu.async_copy(ei_hbm.at[:, pl.ds(base, CHUNK)], sd_b, sems[b])

        def wait(b):
            et_b, sd_b = bufs[b]
            pltpu.make_async_copy(et_hbm.at[pl.ds(0, CHUNK)], et_b,
                                  sems[b]).wait()
            pltpu.make_async_copy(ei_hbm.at[:, pl.ds(0, CHUNK)], sd_b,
                                  sems[b]).wait()

        lane = lax.iota(jnp.int32, L)

        def compute(slot, b, accs):
            et_b, sd_b = bufs[b]
            valid = (chunk_id(slot) == wid + slot * NUM_WORKERS)
            vmask = jnp.where(valid, 1.0, 0.0).astype(jnp.float32)
            vmask = jnp.broadcast_to(vmask, (L,))

            def vbody(i, accs):
                out = []
                for u in range(NACC):
                    j = i * NACC + u
                    sl = pl.ds(j * L, L)
                    col = j * L + lane
                    et = et_b[sl]
                    s = plsc.load_gather(sd_b, [jnp.zeros((L,), jnp.int32),
                                                col])
                    d = plsc.load_gather(sd_b, [jnp.ones((L,), jnp.int32),
                                                col])
                    a = plsc.load_gather(nt_v, [s])
                    bb = plsc.load_gather(nt_v, [d])
                    enc = et * 9 + a * 3 + bb
                    wv = plsc.load_gather(w_v, [enc])
                    out.append(accs[u] + wv * vmask)
                return tuple(out)

            return plsc.parallel_loop(
                0, (CHUNK // L) // NACC, carry=accs, unroll=UNROLL)(vbody)

        for b in range(NBUF):
            start(b, b)

        @pl.when(sid == 0)
        def _():
            pltpu.sync_copy(nt_hbm, nt_sp)
        plsc.subcore_barrier()
        pltpu.sync_copy(nt_sp, nt_v)
        pltpu.sync_copy(w_hbm, w_v)

        def ring_body(kk, accs):
            slot = kk * NBUF
            for b in range(NBUF):
                wait(b)
                accs = compute(slot + b, b, accs)
                start(slot + b + NBUF, b)
            return accs

        zero = jnp.zeros((L,), jnp.float32)
        accs = lax.fori_loop(0, SLOTS // NBUF, ring_body, (zero,) * NACC)
        for b in range(NBUF):
            wait(b)
        acc = accs[0]
        for u in range(1, NACC):
            acc = acc + accs[u]
        acc_v[...] = acc
        pltpu.sync_copy(acc_v, out_hbm.at[wid])

    return k(node_type, edge_type, edge_index, w_flat)


def kernel(node_type, edge_type, edge_index, scorer_weight):
    w_flat = scorer_weight.reshape(TABLE_ROWS)
    partials = _sc_graph_counter(node_type, edge_type, edge_index, w_flat)
    return jnp.sum(partials).reshape(1, 1)
